# baseline (speedup 1.0000x reference)
import jax
import jax.numpy as jnp
from jax import lax
from jax.experimental import pallas as pl

_B, _C, _H, _W = 8, 64, 32, 32
_K = 8192
_N = _B * _H * _W
_HW = _H * _W
_TEMPERATURE = 0.1

_BN = 128
_NPB = _HW // _BN


def _vq_body(flat_ref, code_ref, dist_ref, assign_ref, q_ref):
    f = flat_ref[...]
    cb = code_ref[...]
    fn = f / jnp.maximum(jnp.sqrt(jnp.sum(f * f, axis=1, keepdims=True)), 1e-12)
    cn = cb / jnp.maximum(jnp.sqrt(jnp.sum(cb * cb, axis=1, keepdims=True)), 1e-12)
    fsq = jnp.sum(fn * fn, axis=1, keepdims=True)
    csq = jnp.sum(cn * cn, axis=1, keepdims=True)

    g_nk = lax.dot_general(fn, cn, (((1,), (1,)), ((), ())),
                           preferred_element_type=jnp.float32)
    dist_ref[...] = fsq + jnp.transpose(csq) - 2.0 * g_nk

    g_kn = lax.dot_general(cn, fn, (((1,), (1,)), ((), ())),
                           preferred_element_type=jnp.float32)
    dist_t = csq + jnp.transpose(fsq) - 2.0 * g_kn

    dmin = jnp.min(dist_t, axis=0, keepdims=True)
    e1 = jnp.exp(dmin - dist_t)
    s1 = jnp.sum(e1, axis=0, keepdims=True)
    q_t = lax.dot_general(cn, e1, (((0,), (0,)), ((), ())),
                          preferred_element_type=jnp.float32)
    q_ref[...] = (q_t / s1)[None]

    e2 = jnp.exp((dmin - dist_t) * (1.0 / _TEMPERATURE))
    s2 = jnp.sum(e2, axis=0, keepdims=True)
    assign_ref[...] = (e2 / s2)[None]


def kernel(x, codebook, cur_iter):
    del cur_iter
    flat = jnp.transpose(x, (0, 2, 3, 1)).reshape(_N, _C)

    dist, assign_t, q_t = pl.pallas_call(
        _vq_body,
        grid=(_N // _BN,),
        in_specs=[
            pl.BlockSpec((_BN, _C), lambda i: (i, 0)),
            pl.BlockSpec((_K, _C), lambda i: (0, 0)),
        ],
        out_specs=[
            pl.BlockSpec((_BN, _K), lambda i: (i, 0)),
            pl.BlockSpec((1, _K, _BN), lambda i: (i // _NPB, 0, i % _NPB)),
            pl.BlockSpec((1, _C, _BN), lambda i: (i // _NPB, 0, i % _NPB)),
        ],
        out_shape=[
            jax.ShapeDtypeStruct((_N, _K), jnp.float32),
            jax.ShapeDtypeStruct((_B, _K, _HW), jnp.float32),
            jax.ShapeDtypeStruct((_B, _C, _HW), jnp.float32),
        ],
    )(flat, codebook)

    q_feat = q_t.reshape(_B, _C, _H, _W)
    assignment = assign_t.reshape(_B, _K, _H, _W)
    return q_feat, assignment, dist


# speedup vs baseline: 1.5522x; 1.5522x over previous
import jax
import jax.numpy as jnp
from jax import lax
from jax.experimental import pallas as pl
from jax.experimental.pallas import tpu as pltpu

_B, _C, _H, _W = 8, 64, 32, 32
_K = 8192
_N = _B * _H * _W
_HW = _H * _W
_INV_T = 10.0

_BN = 128
_NPB = _HW // _BN


def _vq_body(flat_ref, code_ref, dist_ref, assign_ref, q_ref,
             cn_ref, cnt_ref, csq_ref, csqt_ref):
    @pl.when(pl.program_id(0) == 0)
    def _prologue():
        cb = code_ref[...]
        cn = cb / jnp.maximum(jnp.sqrt(jnp.sum(cb * cb, axis=1, keepdims=True)),
                              1e-12)
        cn_ref[...] = cn
        cnt_ref[...] = jnp.transpose(cn)
        csq = jnp.sum(cn * cn, axis=1, keepdims=True)
        csq_ref[...] = csq
        csqt_ref[...] = jnp.transpose(csq)

    cn = cn_ref[...]
    cnt = cnt_ref[...]
    csq = csq_ref[...]
    csqt = csqt_ref[...]

    f = flat_ref[...]
    fn = f / jnp.maximum(jnp.sqrt(jnp.sum(f * f, axis=1, keepdims=True)), 1e-12)
    fsq = jnp.sum(fn * fn, axis=1, keepdims=True)
    fsqt = jnp.transpose(fsq)
    fnt = jnp.transpose(fn)

    g_nk = lax.dot_general(fn, cnt, (((1,), (0,)), ((), ())),
                           preferred_element_type=jnp.float32)
    dist_ref[...] = (fsq + csqt) - 2.0 * g_nk

    g_kn = lax.dot_general(cn, fnt, (((1,), (0,)), ((), ())),
                           preferred_element_type=jnp.float32)

    e1 = jnp.exp(2.0 * g_kn - (csq + fsqt))
    s1 = jnp.sum(e1, axis=0, keepdims=True)
    q_t = lax.dot_general(cnt, e1, (((1,), (0,)), ((), ())),
                          preferred_element_type=jnp.float32)
    q_ref[...] = (q_t / s1)[None]

    e2 = jnp.exp((2.0 * _INV_T) * g_kn - (_INV_T * csq + _INV_T * fsqt))
    s2 = jnp.sum(e2, axis=0, keepdims=True)
    assign_ref[...] = (e2 / s2)[None]


def kernel(x, codebook, cur_iter):
    del cur_iter
    flat = jnp.transpose(x, (0, 2, 3, 1)).reshape(_N, _C)

    dist, assign_t, q_t = pl.pallas_call(
        _vq_body,
        grid=(_N // _BN,),
        in_specs=[
            pl.BlockSpec((_BN, _C), lambda i: (i, 0)),
            pl.BlockSpec((_K, _C), lambda i: (0, 0)),
        ],
        out_specs=[
            pl.BlockSpec((_BN, _K), lambda i: (i, 0)),
            pl.BlockSpec((1, _K, _BN), lambda i: (i // _NPB, 0, i % _NPB)),
            pl.BlockSpec((1, _C, _BN), lambda i: (i // _NPB, 0, i % _NPB)),
        ],
        out_shape=[
            jax.ShapeDtypeStruct((_N, _K), jnp.float32),
            jax.ShapeDtypeStruct((_B, _K, _HW), jnp.float32),
            jax.ShapeDtypeStruct((_B, _C, _HW), jnp.float32),
        ],
        scratch_shapes=[
            pltpu.VMEM((_K, _C), jnp.float32),
            pltpu.VMEM((_C, _K), jnp.float32),
            pltpu.VMEM((_K, 1), jnp.float32),
            pltpu.VMEM((1, _K), jnp.float32),
        ],
    )(flat, codebook)

    q_feat = q_t.reshape(_B, _C, _H, _W)
    assignment = assign_t.reshape(_B, _K, _H, _W)
    return q_feat, assignment, dist


# speedup vs baseline: 1.7928x; 1.1550x over previous
import jax
import jax.numpy as jnp
from jax import lax
from jax.experimental import pallas as pl
from jax.experimental.pallas import tpu as pltpu

_B, _C, _H, _W = 8, 64, 32, 32
_K = 8192
_N = _B * _H * _W
_HW = _H * _W
_INV_T = 10.0

_BN = 128
_NPB = _HW // _BN


def _vq_body(flat_ref, code_ref, dist_ref, assign_ref, q_ref,
             cnta_ref, cna_ref):
    @pl.when(pl.program_id(0) == 0)
    def _prologue():
        cb = code_ref[...]
        cn = cb / jnp.maximum(jnp.sqrt(jnp.sum(cb * cb, axis=1, keepdims=True)),
                              1e-12)
        csq = jnp.sum(cn * cn, axis=1, keepdims=True)
        ones_k = jnp.ones((_K, 1), jnp.float32)
        cnta_ref[...] = jnp.transpose(
            jnp.concatenate([cn, ones_k, csq], axis=1))
        cna_ref[...] = jnp.concatenate([2.0 * cn, csq, ones_k], axis=1)

    f = flat_ref[...]
    fn = f / jnp.maximum(jnp.sqrt(jnp.sum(f * f, axis=1, keepdims=True)), 1e-12)
    fsq = jnp.sum(fn * fn, axis=1, keepdims=True)
    ones_n = jnp.ones((_BN, 1), jnp.float32)

    fn_aug = jnp.concatenate([-2.0 * fn, fsq, ones_n], axis=1)
    dist_ref[...] = lax.dot_general(fn_aug, cnta_ref[...],
                                    (((1,), (0,)), ((), ())),
                                    preferred_element_type=jnp.float32)

    fnt_aug = jnp.transpose(
        jnp.concatenate([fn, -ones_n, -fsq], axis=1))
    arg = lax.dot_general(cna_ref[...], fnt_aug, (((1,), (0,)), ((), ())),
                          preferred_element_type=jnp.float32)

    e1 = jnp.exp(arg)
    qs = lax.dot_general(cnta_ref[0:65, :], e1, (((1,), (0,)), ((), ())),
                         preferred_element_type=jnp.float32)
    q_ref[...] = (qs[0:64, :] / qs[64:65, :])[None]

    e2 = jnp.exp(_INV_T * arg)
    s2 = jnp.sum(e2, axis=0, keepdims=True)
    assign_ref[...] = (e2 / s2)[None]


def kernel(x, codebook, cur_iter):
    del cur_iter
    flat = jnp.transpose(x, (0, 2, 3, 1)).reshape(_N, _C)

    dist, assign_t, q_t = pl.pallas_call(
        _vq_body,
        grid=(_N // _BN,),
        in_specs=[
            pl.BlockSpec((_BN, _C), lambda i: (i, 0)),
            pl.BlockSpec((_K, _C), lambda i: (0, 0)),
        ],
        out_specs=[
            pl.BlockSpec((_BN, _K), lambda i: (i, 0)),
            pl.BlockSpec((1, _K, _BN), lambda i: (i // _NPB, 0, i % _NPB)),
            pl.BlockSpec((1, _C, _BN), lambda i: (i // _NPB, 0, i % _NPB)),
        ],
        out_shape=[
            jax.ShapeDtypeStruct((_N, _K), jnp.float32),
            jax.ShapeDtypeStruct((_B, _K, _HW), jnp.float32),
            jax.ShapeDtypeStruct((_B, _C, _HW), jnp.float32),
        ],
        scratch_shapes=[
            pltpu.VMEM((66, _K), jnp.float32),
            pltpu.VMEM((_K, 66), jnp.float32),
        ],
    )(flat, codebook)

    q_feat = q_t.reshape(_B, _C, _H, _W)
    assignment = assign_t.reshape(_B, _K, _H, _W)
    return q_feat, assignment, dist


# speedup vs baseline: 1.8236x; 1.0172x over previous
import jax
import jax.numpy as jnp
from jax import lax
from jax.experimental import pallas as pl
from jax.experimental.pallas import tpu as pltpu

_B, _C, _H, _W = 8, 64, 32, 32
_K = 8192
_N = _B * _H * _W
_HW = _H * _W
_INV_T = 10.0

_BN = 256
_NPB = _HW // _BN


def _vq_body(xf_ref, code_ref, dist_ref, assign_ref, q_ref, cnta_ref):
    @pl.when(pl.program_id(0) == 0)
    def _prologue():
        cbt = code_ref[...]
        s = jnp.sum(cbt * cbt, axis=0, keepdims=True)
        cnt = cbt / jnp.maximum(jnp.sqrt(s), 1e-12)
        csqt = jnp.sum(cnt * cnt, axis=0, keepdims=True)
        cnta_ref[...] = jnp.concatenate(
            [cnt, jnp.ones((1, _K), jnp.float32), csqt], axis=0)

    ft = xf_ref[0]
    s = jnp.sum(ft * ft, axis=0, keepdims=True)
    r = 1.0 / jnp.maximum(jnp.sqrt(s), 1e-12)
    fnt = ft * r
    fnt2 = fnt + fnt
    fsqt = jnp.sum(fnt * fnt, axis=0, keepdims=True)
    ones_n = jnp.ones((1, _BN), jnp.float32)

    m = lax.dot_general(cnta_ref[0:64, :], fnt2, (((0,), (0,)), ((), ())),
                        preferred_element_type=jnp.float32)

    fa_t = jnp.concatenate([-fnt2, fsqt, ones_n], axis=0)
    dist_ref[...] = lax.dot_general(jnp.transpose(fa_t), cnta_ref[...],
                                    (((1,), (0,)), ((), ())),
                                    preferred_element_type=jnp.float32)

    e2 = jnp.exp(_INV_T * m)
    s2 = jnp.sum(e2, axis=0, keepdims=True)
    assign_ref[...] = (e2 / s2)[None]

    e1 = jnp.exp(m)
    qs = lax.dot_general(cnta_ref[0:65, :], e1, (((1,), (0,)), ((), ())),
                         preferred_element_type=jnp.float32)
    q_ref[...] = (qs[0:64, :] / qs[64:65, :])[None]


def kernel(x, codebook, cur_iter):
    del cur_iter
    xf = x.reshape(_B, _C, _HW)
    cbt = jnp.transpose(codebook)

    dist, assign_t, q_t = pl.pallas_call(
        _vq_body,
        grid=(_N // _BN,),
        in_specs=[
            pl.BlockSpec((1, _C, _BN), lambda i: (i // _NPB, 0, i % _NPB)),
            pl.BlockSpec((_C, _K), lambda i: (0, 0)),
        ],
        out_specs=[
            pl.BlockSpec((_BN, _K), lambda i: (i, 0)),
            pl.BlockSpec((1, _K, _BN), lambda i: (i // _NPB, 0, i % _NPB)),
            pl.BlockSpec((1, _C, _BN), lambda i: (i // _NPB, 0, i % _NPB)),
        ],
        out_shape=[
            jax.ShapeDtypeStruct((_N, _K), jnp.float32),
            jax.ShapeDtypeStruct((_B, _K, _HW), jnp.float32),
            jax.ShapeDtypeStruct((_B, _C, _HW), jnp.float32),
        ],
        scratch_shapes=[
            pltpu.VMEM((66, _K), jnp.float32),
        ],
    )(xf, cbt)

    q_feat = q_t.reshape(_B, _C, _H, _W)
    assignment = assign_t.reshape(_B, _K, _H, _W)
    return q_feat, assignment, dist
